# baseline (speedup 1.0000x reference)
import functools

import jax
import jax.numpy as jnp
from jax import lax
from jax.experimental import pallas as pl
from jax.experimental.pallas import tpu as pltpu
from jax.experimental.pallas import tpu_sc as plsc

N_NODES = 10000
NT = 8192
E = 160000
L = 3
D = 128
H = 4
HD = H * D
NSLICE = 64
TPS = NT // NSLICE
NWORK = 32
SPW = NSLICE // NWORK
EP_ROWS = 1280


def _bounds_body(t0_ref, t1_ref, rp0_ref, rp1_ref):
    krow = lax.broadcasted_iota(jnp.int32, (128, 128), 0)

    def one(tref, rpref):
        def body(r, acc):
            row = tref[pl.ds(r, 1), :]
            sid = row >> 7
            return acc + (krow == sid).astype(jnp.float32)

        hist = lax.fori_loop(0, EP_ROWS, body, jnp.zeros((128, 128), jnp.float32))
        hist_row = jnp.sum(hist, axis=1)[None, :]
        s_ids = lax.broadcasted_iota(jnp.int32, (128, 128), 1)
        mask = (s_ids < krow).astype(jnp.float32)
        rp = jnp.sum(mask * hist_row, axis=1, keepdims=True)
        rpref[...] = rp.astype(jnp.int32)

    one(t0_ref, rp0_ref)
    one(t1_ref, rp1_ref)


def _bounds(tgt0, tgt1):
    pad = EP_ROWS * 128 - E
    big = jnp.full((pad,), jnp.int32(1 << 30), jnp.int32)
    t0 = jnp.concatenate([tgt0, big]).reshape(EP_ROWS, 128)
    t1 = jnp.concatenate([tgt1, big]).reshape(EP_ROWS, 128)
    rp0, rp1 = pl.pallas_call(
        _bounds_body,
        out_shape=(
            jax.ShapeDtypeStruct((128, 1), jnp.int32),
            jax.ShapeDtypeStruct((128, 1), jnp.int32),
        ),
    )(t0, t1)
    return rp0.reshape(128), rp1.reshape(128)


def _proj_body(big_ref, w8_ref, out_ref):
    out_ref[...] = jnp.dot(big_ref[...], w8_ref[...],
                           preferred_element_type=jnp.float32)


def _proj(big, w8):
    NB = 20
    BS = 2 * N_NODES // NB
    return pl.pallas_call(
        _proj_body,
        grid=(NB,),
        in_specs=[
            pl.BlockSpec((BS, D), lambda i: (i, 0)),
            pl.BlockSpec((D, 8), lambda i: (0, 0)),
        ],
        out_specs=pl.BlockSpec((BS, 8), lambda i: (i, 0)),
        out_shape=jax.ShapeDtypeStruct((2 * N_NODES, 8), jnp.float32),
    )(big, w8)


NCHUNK = E // 16


def _sc_agg_body(big, ft2, idxc, tgt, nl, rp, zacc,
                 hp_out,
                 acc_v, den_v, a1g_v, nl_v, rp_v,
                 idxc_v0, idxc_v1, tgt_v0, tgt_v1, big_v0, big_v1,
                 ft2g_v0, ft2g_v1, g_v,
                 isem0, isem1, tsem0, tsem1, gsem0, gsem1, fsem0, fsem1,
                 hsem0, hsem1):
    cid = lax.axis_index("c")
    sid = lax.axis_index("s")
    wid = cid * 16 + sid

    idxc_vs = (idxc_v0, idxc_v1)
    tgt_vs = (tgt_v0, tgt_v1)
    big_vs = (big_v0, big_v1)
    ft2g_vs = (ft2g_v0, ft2g_v1)
    isems = (isem0, isem1)
    tsems = (tsem0, tsem1)
    gsems = (gsem0, gsem1)
    fsems = (fsem0, fsem1)
    hsems = (hsem0, hsem1)

    pltpu.sync_copy(rp, rp_v)

    lane = lax.broadcasted_iota(jnp.int32, (16,), 0)
    lane4f = (lane < 4).astype(jnp.float32)
    row4 = lane * 4
    topic_off = jnp.where(lane % 4 == 3, N_NODES, 0)
    third = jnp.float32(1.0 / 3.0)
    zero16 = jnp.zeros((16,), jnp.float32)

    def dma_idx(c, b):
        cc = jnp.minimum(c, NCHUNK - 1)
        pltpu.async_copy(idxc.at[pl.ds(cc * 64, 64)], idxc_vs[b], isems[b])
        pltpu.async_copy(tgt.at[pl.ds(cc * 16, 16)], tgt_vs[b].at[pl.ds(0, 16)],
                         tsems[b])

    def wait_idx(b):
        pltpu.make_async_copy(idxc.at[pl.ds(0, 64)], idxc_vs[b], isems[b]).wait()
        pltpu.make_async_copy(tgt.at[pl.ds(0, 16)], tgt_vs[b].at[pl.ds(0, 16)],
                              tsems[b]).wait()

    def fix_idx(b):
        for q in range(4):
            dq = pl.ds(16 * q, 16)
            idxc_vs[b][dq] = idxc_vs[b][dq] + topic_off

    def dma_gather(b):
        pltpu.async_copy(big.at[idxc_vs[b].at[pl.ds(0, 32)]],
                         big_vs[b].at[pl.ds(0, 32), :], gsems[b])
        pltpu.async_copy(big.at[idxc_vs[b].at[pl.ds(32, 32)]],
                         big_vs[b].at[pl.ds(32, 32), :], hsems[b])
        pltpu.async_copy(ft2.at[idxc_vs[b]], ft2g_vs[b], fsems[b])

    def wait_gather(b):
        pltpu.make_async_copy(big.at[idxc_vs[b].at[pl.ds(0, 32)]],
                              big_vs[b].at[pl.ds(0, 32), :], gsems[b]).wait()
        pltpu.make_async_copy(big.at[idxc_vs[b].at[pl.ds(32, 32)]],
                              big_vs[b].at[pl.ds(32, 32), :], hsems[b]).wait()
        pltpu.make_async_copy(ft2.at[idxc_vs[b]], ft2g_vs[b], fsems[b]).wait()

    def slice_body(r, _):
        k = wid * SPW + r
        t0 = k * TPS

        pltpu.sync_copy(zacc, acc_v)

        def zero_body(t, _):
            den_v[t, :] = zero16
            return 0

        lax.fori_loop(0, TPS, zero_body, 0)

        pltpu.sync_copy(nl.at[pl.ds(t0, TPS)], nl_v)
        pltpu.async_copy(ft2.at[nl_v], a1g_v, gsem0).wait()

        rpv = rp_v[pl.ds(k, 16)]
        e0 = rpv[0]
        e1 = rpv[1]
        c0 = e0 // 16
        c1 = (e1 + 15) // 16

        def compute(c, b):
            base = c * 16
            tgt_b = tgt_vs[b]
            big_b = big_vs[b]
            tvec = tgt_b[pl.ds(0, 16)]
            t_c16 = jnp.minimum(jnp.maximum(tvec - t0, 0), TPS - 1)
            ev = lane + base
            vf = ((ev >= e0) & (ev < e1)).astype(jnp.float32)
            for h in range(H):
                hv = jnp.full((16,), h, jnp.int32)
                a1vec = plsc.load_gather(a1g_v, [t_c16, hv + 4])
                s0 = plsc.load_gather(ft2g_vs[b], [row4, hv])
                s1 = plsc.load_gather(ft2g_vs[b], [row4 + 1, hv])
                s2 = plsc.load_gather(ft2g_vs[b], [row4 + 2, hv])
                s3 = plsc.load_gather(ft2g_vs[b], [row4 + 3, hv])
                a = a1vec + (s0 + s1 + s2) * third + s3
                a = jnp.maximum(a, jnp.float32(0.01) * a)
                g = jnp.exp(a) * vf
                plsc.store_scatter(g_v, [lane, hv], g)

            def edge_body(e, _):
                b4 = e * 4
                t = tgt_b[pl.ds(e, 16)][0] - t0
                t_c = jnp.minimum(jnp.maximum(t, 0), TPS - 1)
                grow = g_v[e, :]
                plsc.addupdate(den_v.at[t_c, :], grow * lane4f)
                gb = [zero16 + grow[h] for h in range(H)]
                for j in range(8):
                    dj = pl.ds(16 * j, 16)
                    hj = (big_b[b4, dj] + big_b[b4 + 1, dj] + big_b[b4 + 2, dj]) \
                        * third + big_b[b4 + 3, dj]
                    for h in range(H):
                        col = 128 * h + 16 * j
                        plsc.addupdate(acc_v.at[t_c, pl.ds(col, 16)], gb[h] * hj)
                return 0

            lax.fori_loop(0, 16, edge_body, 0)

        dma_idx(c0, 0)
        wait_idx(0)
        fix_idx(0)
        dma_gather(0)
        npairs = (c1 - c0 + 1) // 2

        def pair_body(i, _):
            c = c0 + 2 * i
            dma_idx(c + 1, 1)
            wait_idx(1)
            fix_idx(1)
            wait_gather(0)
            dma_gather(1)
            compute(c, 0)
            dma_idx(c + 2, 0)
            wait_idx(0)
            fix_idx(0)
            wait_gather(1)
            dma_gather(0)
            compute(c + 1, 1)
            return 0

        lax.fori_loop(0, npairs, pair_body, 0)
        wait_gather(0)

        def fin_body(t, _):
            drow = den_v[t, :]
            for h in range(H):
                dspl = zero16 + (drow[h] + jnp.float32(1e-9))
                for j in range(8):
                    col = 128 * h + 16 * j
                    v = acc_v[t, pl.ds(col, 16)] / dspl
                    v = jnp.where(v > 0, v, jnp.exp(v) - jnp.float32(1.0))
                    acc_v[t, pl.ds(col, 16)] = v
            return 0

        lax.fori_loop(0, TPS, fin_body, 0)
        pltpu.sync_copy(acc_v, hp_out.at[pl.ds(t0, TPS), :])
        return 0

    lax.fori_loop(0, SPW, slice_body, 0)


def _sc_agg(big, ft2, idxc, tgt, nl, rp, zacc):
    mesh = plsc.VectorSubcoreMesh(core_axis_name="c", subcore_axis_name="s")
    f = pl.kernel(
        _sc_agg_body,
        out_type=jax.ShapeDtypeStruct((NT, HD), jnp.float32),
        mesh=mesh,
        compiler_params=pltpu.CompilerParams(needs_layout_passes=False,
                                             use_tc_tiling_on_sc=False),
        scratch_types=[
            pltpu.VMEM((TPS, HD), jnp.float32),
            pltpu.VMEM((TPS, 16), jnp.float32),
            pltpu.VMEM((TPS, 8), jnp.float32),
            pltpu.VMEM((TPS,), jnp.int32),
            pltpu.VMEM((128,), jnp.int32),
            pltpu.VMEM((64,), jnp.int32),
            pltpu.VMEM((64,), jnp.int32),
            pltpu.VMEM((32,), jnp.int32),
            pltpu.VMEM((32,), jnp.int32),
            pltpu.VMEM((64, D), jnp.float32),
            pltpu.VMEM((64, D), jnp.float32),
            pltpu.VMEM((64, 8), jnp.float32),
            pltpu.VMEM((64, 8), jnp.float32),
            pltpu.VMEM((16, 16), jnp.float32),
        ] + [pltpu.SemaphoreType.DMA] * 10,
    )
    return f(big, ft2, idxc, tgt, nl, rp, zacc)


def _scores_body(hp0_ref, hp1_ref, w1_ref, b1_ref, w2_ref, s0_ref, s1_ref):
    i = pl.program_id(0)

    @pl.when(i == 0)
    def _():
        s0_ref[0, 0] = jnp.float32(0.0)
        s1_ref[0, 0] = jnp.float32(0.0)

    w1 = w1_ref[...]
    b1 = b1_ref[...]
    w2 = w2_ref[...]
    z0 = jnp.tanh(jnp.dot(hp0_ref[...], w1, preferred_element_type=jnp.float32) + b1)
    z1 = jnp.tanh(jnp.dot(hp1_ref[...], w1, preferred_element_type=jnp.float32) + b1)
    s0_ref[0, 0] += jnp.sum(z0 * w2)
    s1_ref[0, 0] += jnp.sum(z1 * w2)


def _scores(hp0, hp1, fc1_w, fc1_b, fc2_w):
    BS = 512
    nb = NT // BS
    s0, s1 = pl.pallas_call(
        _scores_body,
        grid=(nb,),
        in_specs=[
            pl.BlockSpec((BS, HD), lambda i: (i, 0)),
            pl.BlockSpec((BS, HD), lambda i: (i, 0)),
            pl.BlockSpec((HD, 128), lambda i: (0, 0)),
            pl.BlockSpec((1, 128), lambda i: (0, 0)),
            pl.BlockSpec((1, 128), lambda i: (0, 0)),
        ],
        out_specs=(
            pl.BlockSpec((1, 1), lambda i: (0, 0), memory_space=pltpu.SMEM),
            pl.BlockSpec((1, 1), lambda i: (0, 0), memory_space=pltpu.SMEM),
        ),
        out_shape=(
            jax.ShapeDtypeStruct((1, 1), jnp.float32),
            jax.ShapeDtypeStruct((1, 1), jnp.float32),
        ),
    )(hp0, hp1, fc1_w, fc1_b.reshape(1, 128), fc2_w.reshape(1, 128))
    return s0, s1


def _combine_body(hp0_ref, hp1_ref, wu_ref, bu_ref, s0_ref, s1_ref,
                  hu_ref, lg_ref, beta_ref):
    i = pl.program_id(0)
    dlt = (s1_ref[0, 0] - s0_ref[0, 0]) / jnp.float32(NT)
    b0 = jnp.float32(1.0) / (jnp.float32(1.0) + jnp.exp(dlt))
    b1 = jnp.float32(1.0) - b0

    @pl.when(i == 0)
    def _():
        col = lax.broadcasted_iota(jnp.int32, (1, 128), 1)
        beta_ref[...] = jnp.where(col == 0, b0, jnp.where(col == 1, b1, 0.0))

    hu = b0 * hp0_ref[...] + b1 * hp1_ref[...]
    hu_ref[...] = hu
    lg_ref[...] = jnp.dot(hu, wu_ref[...], preferred_element_type=jnp.float32) \
        + bu_ref[...]


def _combine(hp0, hp1, fc_user_w, fc_user_b, s0, s1):
    BS = 512
    nb = NT // BS
    return pl.pallas_call(
        _combine_body,
        grid=(nb,),
        in_specs=[
            pl.BlockSpec((BS, HD), lambda i: (i, 0)),
            pl.BlockSpec((BS, HD), lambda i: (i, 0)),
            pl.BlockSpec((HD, D), lambda i: (0, 0)),
            pl.BlockSpec((1, D), lambda i: (0, 0)),
            pl.BlockSpec(memory_space=pltpu.SMEM),
            pl.BlockSpec(memory_space=pltpu.SMEM),
        ],
        out_specs=(
            pl.BlockSpec((BS, HD), lambda i: (i, 0)),
            pl.BlockSpec((BS, D), lambda i: (i, 0)),
            pl.BlockSpec((1, 128), lambda i: (0, 0)),
        ),
        out_shape=(
            jax.ShapeDtypeStruct((NT, HD), jnp.float32),
            jax.ShapeDtypeStruct((NT, D), jnp.float32),
            jax.ShapeDtypeStruct((1, 128), jnp.float32),
        ),
    )(hp0, hp1, fc_user_w, fc_user_b.reshape(1, D), s0, s1)


@jax.jit
def kernel(features, topic, type_mask, edge_metapath_indices_0,
           edge_metapath_indices_1, edge_metapath_text_indices_0,
           edge_metapath_text_indices_1, target_idx_0, target_idx_1,
           node_list_0, node_list_1, attn1, attn2, fc1_w, fc1_b, fc2_w,
           fc_user_w, fc_user_b):
    del type_mask
    i32 = jnp.int32
    idxc0 = jnp.concatenate(
        [edge_metapath_indices_0.astype(i32),
         edge_metapath_text_indices_0.astype(i32)[:, None]], axis=1).reshape(-1)
    idxc1 = jnp.concatenate(
        [edge_metapath_indices_1.astype(i32),
         edge_metapath_text_indices_1.astype(i32)[:, None]], axis=1).reshape(-1)
    tgt0 = target_idx_0.astype(i32)
    tgt1 = target_idx_1.astype(i32)
    nl0 = node_list_0.astype(i32)
    nl1 = node_list_1.astype(i32)
    big = jnp.concatenate([features, topic], axis=0)
    w8 = jnp.concatenate([attn2.T, attn1], axis=1)
    zacc = jnp.zeros((TPS, HD), jnp.float32)

    ft2 = _proj(big, w8)
    rp0, rp1 = _bounds(tgt0, tgt1)
    hp0 = _sc_agg(big, ft2, idxc0, tgt0, nl0, rp0, zacc)
    hp1 = _sc_agg(big, ft2, idxc1, tgt1, nl1, rp1, zacc)
    s0, s1 = _scores(hp0, hp1, fc1_w, fc1_b, fc2_w)
    h_user, logits, beta_mat = _combine(hp0, hp1, fc_user_w, fc_user_b, s0, s1)
    return h_user, logits, beta_mat[0, :2]


# speedup vs baseline: 1.3785x; 1.3785x over previous
import functools

import jax
import jax.numpy as jnp
from jax import lax
from jax.experimental import pallas as pl
from jax.experimental.pallas import tpu as pltpu
from jax.experimental.pallas import tpu_sc as plsc

N_NODES = 10000
NT = 8192
E = 160000
L = 3
D = 128
H = 4
HD = H * D
NSLICE = 64
TPS = NT // NSLICE
NWORK = 32
SPW = NSLICE // NWORK
EP_ROWS = 1280


def _bounds_body(t0_ref, t1_ref, rp0_ref, rp1_ref):
    krow = lax.broadcasted_iota(jnp.int32, (128, 128), 0)

    def one(tref, rpref):
        def body(r, acc):
            row = tref[pl.ds(r, 1), :]
            sid = row >> 7
            return acc + (krow == sid).astype(jnp.float32)

        hist = lax.fori_loop(0, EP_ROWS, body, jnp.zeros((128, 128), jnp.float32))
        hist_row = jnp.sum(hist, axis=1)[None, :]
        s_ids = lax.broadcasted_iota(jnp.int32, (128, 128), 1)
        mask = (s_ids < krow).astype(jnp.float32)
        rp = jnp.sum(mask * hist_row, axis=1, keepdims=True)
        rpref[...] = rp.astype(jnp.int32)

    one(t0_ref, rp0_ref)
    one(t1_ref, rp1_ref)


def _bounds(tgt0, tgt1):
    pad = EP_ROWS * 128 - E
    big = jnp.full((pad,), jnp.int32(1 << 30), jnp.int32)
    t0 = jnp.concatenate([tgt0, big]).reshape(EP_ROWS, 128)
    t1 = jnp.concatenate([tgt1, big]).reshape(EP_ROWS, 128)
    rp0, rp1 = pl.pallas_call(
        _bounds_body,
        out_shape=(
            jax.ShapeDtypeStruct((128, 1), jnp.int32),
            jax.ShapeDtypeStruct((128, 1), jnp.int32),
        ),
    )(t0, t1)
    return rp0.reshape(128), rp1.reshape(128)


def _proj_body(big_ref, w8_ref, out_ref):
    out_ref[...] = jnp.dot(big_ref[...], w8_ref[...],
                           preferred_element_type=jnp.float32)


def _proj(big, w8):
    NB = 20
    BS = 2 * N_NODES // NB
    return pl.pallas_call(
        _proj_body,
        grid=(NB,),
        in_specs=[
            pl.BlockSpec((BS, D), lambda i: (i, 0)),
            pl.BlockSpec((D, 8), lambda i: (0, 0)),
        ],
        out_specs=pl.BlockSpec((BS, 8), lambda i: (i, 0)),
        out_shape=jax.ShapeDtypeStruct((2 * N_NODES, 8), jnp.float32),
    )(big, w8)


NCHUNK = E // 16


def _sc_agg_body(big, ft2, idxc, tgt, nl, rp, zacc,
                 hp_out,
                 acc_v, den_v, a1g_v, nl_v, rp_v,
                 idxc_v0, idxc_v1, tgt_v0, tgt_v1, big_v0, big_v1,
                 ft2g_v0, ft2g_v1, g_v,
                 isem0, isem1, tsem0, tsem1, gsem0, gsem1, fsem0, fsem1,
                 hsem0, hsem1):
    cid = lax.axis_index("c")
    sid = lax.axis_index("s")
    wid = cid * 16 + sid

    idxc_vs = (idxc_v0, idxc_v1)
    tgt_vs = (tgt_v0, tgt_v1)
    big_vs = (big_v0, big_v1)
    ft2g_vs = (ft2g_v0, ft2g_v1)
    isems = (isem0, isem1)
    tsems = (tsem0, tsem1)
    gsems = (gsem0, gsem1)
    fsems = (fsem0, fsem1)
    hsems = (hsem0, hsem1)

    pltpu.sync_copy(rp, rp_v)

    lane = lax.broadcasted_iota(jnp.int32, (16,), 0)
    lane4f = (lane < 4).astype(jnp.float32)
    row4 = lane * 4
    topic_off = jnp.where(lane % 4 == 3, N_NODES, 0)
    third = jnp.float32(1.0 / 3.0)
    zero16 = jnp.zeros((16,), jnp.float32)

    def dma_idx(c, b):
        cc = jnp.minimum(c, NCHUNK - 1)
        pltpu.async_copy(idxc.at[pl.ds(cc * 64, 64)], idxc_vs[b], isems[b])
        pltpu.async_copy(tgt.at[pl.ds(cc * 16, 16)], tgt_vs[b].at[pl.ds(0, 16)],
                         tsems[b])

    def wait_idx(b):
        pltpu.make_async_copy(idxc.at[pl.ds(0, 64)], idxc_vs[b], isems[b]).wait()
        pltpu.make_async_copy(tgt.at[pl.ds(0, 16)], tgt_vs[b].at[pl.ds(0, 16)],
                              tsems[b]).wait()

    def fix_idx(b):
        for q in range(4):
            dq = pl.ds(16 * q, 16)
            idxc_vs[b][dq] = idxc_vs[b][dq] + topic_off

    def dma_gather(b):
        pltpu.async_copy(big.at[idxc_vs[b].at[pl.ds(0, 32)]],
                         big_vs[b].at[pl.ds(0, 32), :], gsems[b])
        pltpu.async_copy(big.at[idxc_vs[b].at[pl.ds(32, 32)]],
                         big_vs[b].at[pl.ds(32, 32), :], hsems[b])
        pltpu.async_copy(ft2.at[idxc_vs[b]], ft2g_vs[b], fsems[b])

    def wait_gather(b):
        pltpu.make_async_copy(big.at[idxc_vs[b].at[pl.ds(0, 32)]],
                              big_vs[b].at[pl.ds(0, 32), :], gsems[b]).wait()
        pltpu.make_async_copy(big.at[idxc_vs[b].at[pl.ds(32, 32)]],
                              big_vs[b].at[pl.ds(32, 32), :], hsems[b]).wait()
        pltpu.make_async_copy(ft2.at[idxc_vs[b]], ft2g_vs[b], fsems[b]).wait()

    def slice_body(r, _):
        k = wid * SPW + r
        t0 = k * TPS

        pltpu.sync_copy(zacc, acc_v)

        def zero_body(t, _):
            den_v[t, :] = zero16
            return 0

        lax.fori_loop(0, TPS, zero_body, 0)

        pltpu.sync_copy(nl.at[pl.ds(t0, TPS)], nl_v)
        pltpu.async_copy(ft2.at[nl_v], a1g_v, gsem0).wait()

        rpv = rp_v[pl.ds(k, 16)]
        e0 = rpv[0]
        e1 = rpv[1]
        c0 = e0 // 16
        c1 = (e1 + 15) // 16

        def compute(c, b):
            base = c * 16
            tgt_b = tgt_vs[b]
            big_b = big_vs[b]
            tvec = tgt_b[pl.ds(0, 16)]
            t_c16 = jnp.minimum(jnp.maximum(tvec - t0, 0), TPS - 1)
            ev = lane + base
            vf = ((ev >= e0) & (ev < e1)).astype(jnp.float32)
            for h in range(H):
                hv = jnp.full((16,), h, jnp.int32)
                a1vec = plsc.load_gather(a1g_v, [t_c16, hv + 4])
                s0 = plsc.load_gather(ft2g_vs[b], [row4, hv])
                s1 = plsc.load_gather(ft2g_vs[b], [row4 + 1, hv])
                s2 = plsc.load_gather(ft2g_vs[b], [row4 + 2, hv])
                s3 = plsc.load_gather(ft2g_vs[b], [row4 + 3, hv])
                a = a1vec + (s0 + s1 + s2) * third + s3
                a = jnp.maximum(a, jnp.float32(0.01) * a)
                g = jnp.exp(a) * vf
                plsc.store_scatter(g_v, [lane, hv], g)

            def flush(t_cur, regs):
                plsc.addupdate(den_v.at[t_cur, :], regs[32])
                for h in range(H):
                    for j in range(8):
                        col = 128 * h + 16 * j
                        plsc.addupdate(acc_v.at[t_cur, pl.ds(col, 16)],
                                       regs[h * 8 + j])

            def edge_body(e, carry):
                t_cur = carry[0]
                regs = carry[1:]
                b4 = e * 4
                t = tgt_b[pl.ds(e, 16)][0] - t0
                t_c = jnp.minimum(jnp.maximum(t, 0), TPS - 1)
                is_new = (t_c != t_cur) & (t_cur >= 0)

                @pl.when(is_new)
                def _():
                    flush(t_cur, regs)

                newv = jnp.zeros((16,), jnp.bool_) | is_new
                grow = g_v[e, :]
                gb = [zero16 + grow[h] for h in range(H)]
                out = [None] * 33
                for j in range(8):
                    dj = pl.ds(16 * j, 16)
                    hj = (big_b[b4, dj] + big_b[b4 + 1, dj] + big_b[b4 + 2, dj]) \
                        * third + big_b[b4 + 3, dj]
                    for h in range(H):
                        c = gb[h] * hj
                        out[h * 8 + j] = jnp.where(newv, c, regs[h * 8 + j] + c)
                dc = grow * lane4f
                out[32] = jnp.where(newv, dc, regs[32] + dc)
                return (t_c,) + tuple(out)

            init = (jnp.int32(-1),) + tuple([zero16] * 33)
            fin = lax.fori_loop(0, 16, edge_body, init)

            @pl.when(fin[0] >= 0)
            def _():
                flush(fin[0], fin[1:])

        dma_idx(c0, 0)
        wait_idx(0)
        fix_idx(0)
        dma_gather(0)
        npairs = (c1 - c0 + 1) // 2

        def pair_body(i, _):
            c = c0 + 2 * i
            dma_idx(c + 1, 1)
            wait_idx(1)
            fix_idx(1)
            wait_gather(0)
            dma_gather(1)
            compute(c, 0)
            dma_idx(c + 2, 0)
            wait_idx(0)
            fix_idx(0)
            wait_gather(1)
            dma_gather(0)
            compute(c + 1, 1)
            return 0

        lax.fori_loop(0, npairs, pair_body, 0)
        wait_gather(0)

        def fin_body(t, _):
            drow = den_v[t, :]
            for h in range(H):
                dspl = zero16 + (drow[h] + jnp.float32(1e-9))
                for j in range(8):
                    col = 128 * h + 16 * j
                    v = acc_v[t, pl.ds(col, 16)] / dspl
                    v = jnp.where(v > 0, v, jnp.exp(v) - jnp.float32(1.0))
                    acc_v[t, pl.ds(col, 16)] = v
            return 0

        lax.fori_loop(0, TPS, fin_body, 0)
        pltpu.sync_copy(acc_v, hp_out.at[pl.ds(t0, TPS), :])
        return 0

    lax.fori_loop(0, SPW, slice_body, 0)


def _sc_agg(big, ft2, idxc, tgt, nl, rp, zacc):
    mesh = plsc.VectorSubcoreMesh(core_axis_name="c", subcore_axis_name="s")
    f = pl.kernel(
        _sc_agg_body,
        out_type=jax.ShapeDtypeStruct((NT, HD), jnp.float32),
        mesh=mesh,
        compiler_params=pltpu.CompilerParams(needs_layout_passes=False,
                                             use_tc_tiling_on_sc=False),
        scratch_types=[
            pltpu.VMEM((TPS, HD), jnp.float32),
            pltpu.VMEM((TPS, 16), jnp.float32),
            pltpu.VMEM((TPS, 8), jnp.float32),
            pltpu.VMEM((TPS,), jnp.int32),
            pltpu.VMEM((128,), jnp.int32),
            pltpu.VMEM((64,), jnp.int32),
            pltpu.VMEM((64,), jnp.int32),
            pltpu.VMEM((32,), jnp.int32),
            pltpu.VMEM((32,), jnp.int32),
            pltpu.VMEM((64, D), jnp.float32),
            pltpu.VMEM((64, D), jnp.float32),
            pltpu.VMEM((64, 8), jnp.float32),
            pltpu.VMEM((64, 8), jnp.float32),
            pltpu.VMEM((16, 16), jnp.float32),
        ] + [pltpu.SemaphoreType.DMA] * 10,
    )
    return f(big, ft2, idxc, tgt, nl, rp, zacc)


def _scores_body(hp0_ref, hp1_ref, w1_ref, b1_ref, w2_ref, s0_ref, s1_ref):
    i = pl.program_id(0)

    @pl.when(i == 0)
    def _():
        s0_ref[0, 0] = jnp.float32(0.0)
        s1_ref[0, 0] = jnp.float32(0.0)

    w1 = w1_ref[...]
    b1 = b1_ref[...]
    w2 = w2_ref[...]
    z0 = jnp.tanh(jnp.dot(hp0_ref[...], w1, preferred_element_type=jnp.float32) + b1)
    z1 = jnp.tanh(jnp.dot(hp1_ref[...], w1, preferred_element_type=jnp.float32) + b1)
    s0_ref[0, 0] += jnp.sum(z0 * w2)
    s1_ref[0, 0] += jnp.sum(z1 * w2)


def _scores(hp0, hp1, fc1_w, fc1_b, fc2_w):
    BS = 512
    nb = NT // BS
    s0, s1 = pl.pallas_call(
        _scores_body,
        grid=(nb,),
        in_specs=[
            pl.BlockSpec((BS, HD), lambda i: (i, 0)),
            pl.BlockSpec((BS, HD), lambda i: (i, 0)),
            pl.BlockSpec((HD, 128), lambda i: (0, 0)),
            pl.BlockSpec((1, 128), lambda i: (0, 0)),
            pl.BlockSpec((1, 128), lambda i: (0, 0)),
        ],
        out_specs=(
            pl.BlockSpec((1, 1), lambda i: (0, 0), memory_space=pltpu.SMEM),
            pl.BlockSpec((1, 1), lambda i: (0, 0), memory_space=pltpu.SMEM),
        ),
        out_shape=(
            jax.ShapeDtypeStruct((1, 1), jnp.float32),
            jax.ShapeDtypeStruct((1, 1), jnp.float32),
        ),
    )(hp0, hp1, fc1_w, fc1_b.reshape(1, 128), fc2_w.reshape(1, 128))
    return s0, s1


def _combine_body(hp0_ref, hp1_ref, wu_ref, bu_ref, s0_ref, s1_ref,
                  hu_ref, lg_ref, beta_ref):
    i = pl.program_id(0)
    dlt = (s1_ref[0, 0] - s0_ref[0, 0]) / jnp.float32(NT)
    b0 = jnp.float32(1.0) / (jnp.float32(1.0) + jnp.exp(dlt))
    b1 = jnp.float32(1.0) - b0

    @pl.when(i == 0)
    def _():
        col = lax.broadcasted_iota(jnp.int32, (1, 128), 1)
        beta_ref[...] = jnp.where(col == 0, b0, jnp.where(col == 1, b1, 0.0))

    hu = b0 * hp0_ref[...] + b1 * hp1_ref[...]
    hu_ref[...] = hu
    lg_ref[...] = jnp.dot(hu, wu_ref[...], preferred_element_type=jnp.float32) \
        + bu_ref[...]


def _combine(hp0, hp1, fc_user_w, fc_user_b, s0, s1):
    BS = 512
    nb = NT // BS
    return pl.pallas_call(
        _combine_body,
        grid=(nb,),
        in_specs=[
            pl.BlockSpec((BS, HD), lambda i: (i, 0)),
            pl.BlockSpec((BS, HD), lambda i: (i, 0)),
            pl.BlockSpec((HD, D), lambda i: (0, 0)),
            pl.BlockSpec((1, D), lambda i: (0, 0)),
            pl.BlockSpec(memory_space=pltpu.SMEM),
            pl.BlockSpec(memory_space=pltpu.SMEM),
        ],
        out_specs=(
            pl.BlockSpec((BS, HD), lambda i: (i, 0)),
            pl.BlockSpec((BS, D), lambda i: (i, 0)),
            pl.BlockSpec((1, 128), lambda i: (0, 0)),
        ),
        out_shape=(
            jax.ShapeDtypeStruct((NT, HD), jnp.float32),
            jax.ShapeDtypeStruct((NT, D), jnp.float32),
            jax.ShapeDtypeStruct((1, 128), jnp.float32),
        ),
    )(hp0, hp1, fc_user_w, fc_user_b.reshape(1, D), s0, s1)


@jax.jit
def kernel(features, topic, type_mask, edge_metapath_indices_0,
           edge_metapath_indices_1, edge_metapath_text_indices_0,
           edge_metapath_text_indices_1, target_idx_0, target_idx_1,
           node_list_0, node_list_1, attn1, attn2, fc1_w, fc1_b, fc2_w,
           fc_user_w, fc_user_b):
    del type_mask
    i32 = jnp.int32
    idxc0 = jnp.concatenate(
        [edge_metapath_indices_0.astype(i32),
         edge_metapath_text_indices_0.astype(i32)[:, None]], axis=1).reshape(-1)
    idxc1 = jnp.concatenate(
        [edge_metapath_indices_1.astype(i32),
         edge_metapath_text_indices_1.astype(i32)[:, None]], axis=1).reshape(-1)
    tgt0 = target_idx_0.astype(i32)
    tgt1 = target_idx_1.astype(i32)
    nl0 = node_list_0.astype(i32)
    nl1 = node_list_1.astype(i32)
    big = jnp.concatenate([features, topic], axis=0)
    w8 = jnp.concatenate([attn2.T, attn1], axis=1)
    zacc = jnp.zeros((TPS, HD), jnp.float32)

    ft2 = _proj(big, w8)
    rp0, rp1 = _bounds(tgt0, tgt1)
    hp0 = _sc_agg(big, ft2, idxc0, tgt0, nl0, rp0, zacc)
    hp1 = _sc_agg(big, ft2, idxc1, tgt1, nl1, rp1, zacc)
    s0, s1 = _scores(hp0, hp1, fc1_w, fc1_b, fc2_w)
    h_user, logits, beta_mat = _combine(hp0, hp1, fc_user_w, fc_user_b, s0, s1)
    return h_user, logits, beta_mat[0, :2]


# speedup vs baseline: 1.3815x; 1.0022x over previous
import functools

import jax
import jax.numpy as jnp
from jax import lax
from jax.experimental import pallas as pl
from jax.experimental.pallas import tpu as pltpu
from jax.experimental.pallas import tpu_sc as plsc

N_NODES = 10000
NT = 8192
E = 160000
L = 3
D = 128
H = 4
HD = H * D
NSLICE = 64
TPS = NT // NSLICE
NWORK = 32
SPW = NSLICE // NWORK
EP_ROWS = 1280


def _bounds_body(t_ref, rp_ref):
    krow = lax.broadcasted_iota(jnp.int32, (128, 128), 0)

    def body(r, acc):
        row = t_ref[pl.ds(r, 1), :]
        sid = row >> 7
        return acc + (krow == sid).astype(jnp.float32)

    hist = lax.fori_loop(0, EP_ROWS, body, jnp.zeros((128, 128), jnp.float32))
    hist_row = jnp.sum(hist, axis=1)[None, :]
    s_ids = lax.broadcasted_iota(jnp.int32, (128, 128), 1)
    mask = (s_ids < krow).astype(jnp.float32)
    rp = jnp.sum(mask * hist_row, axis=1, keepdims=True)
    rp_ref[...] = rp.astype(jnp.int32)


def _bounds(tgt):
    pad = EP_ROWS * 128 - E
    fill = jnp.full((pad,), jnp.int32(1 << 30), jnp.int32)
    t = jnp.concatenate([tgt, fill]).reshape(EP_ROWS, 128)
    rp = pl.pallas_call(
        _bounds_body,
        out_shape=jax.ShapeDtypeStruct((128, 1), jnp.int32),
    )(t)
    return rp.reshape(128)


def _proj_body(big_ref, w8_ref, out_ref):
    out_ref[...] = jnp.dot(big_ref[...], w8_ref[...],
                           preferred_element_type=jnp.float32)


def _proj(big, w8):
    NB = 20
    BS = 2 * N_NODES // NB
    return pl.pallas_call(
        _proj_body,
        grid=(NB,),
        in_specs=[
            pl.BlockSpec((BS, D), lambda i: (i, 0)),
            pl.BlockSpec((D, 8), lambda i: (0, 0)),
        ],
        out_specs=pl.BlockSpec((BS, 8), lambda i: (i, 0)),
        out_shape=jax.ShapeDtypeStruct((2 * N_NODES, 8), jnp.float32),
    )(big, w8)


NCHUNK = E // 16


def _sc_agg_body(big, ft2, idxc, tgt, nl, rp, zacc,
                 hp_out,
                 acc_v, den_v, a1g_v, nl_v, rp_v,
                 idxc_v0, idxc_v1, tgt_v0, tgt_v1, big_v0, big_v1,
                 ft2g_v0, ft2g_v1, g_v,
                 isem0, isem1, tsem0, tsem1, gsem0, gsem1, fsem0, fsem1,
                 hsem0, hsem1):
    cid = lax.axis_index("c")
    sid = lax.axis_index("s")
    wid = cid * 16 + sid

    idxc_vs = (idxc_v0, idxc_v1)
    tgt_vs = (tgt_v0, tgt_v1)
    big_vs = (big_v0, big_v1)
    ft2g_vs = (ft2g_v0, ft2g_v1)
    isems = (isem0, isem1)
    tsems = (tsem0, tsem1)
    gsems = (gsem0, gsem1)
    fsems = (fsem0, fsem1)
    hsems = (hsem0, hsem1)

    pltpu.sync_copy(rp, rp_v)

    lane = lax.broadcasted_iota(jnp.int32, (16,), 0)
    lane4f = (lane < 4).astype(jnp.float32)
    row4 = lane * 4
    topic_off = jnp.where(lane % 4 == 3, N_NODES, 0)
    third = jnp.float32(1.0 / 3.0)
    zero16 = jnp.zeros((16,), jnp.float32)

    def dma_idx(c, b):
        cc = jnp.minimum(c, NCHUNK - 1)
        pltpu.async_copy(idxc.at[pl.ds(cc * 64, 64)], idxc_vs[b], isems[b])
        pltpu.async_copy(tgt.at[pl.ds(cc * 16, 16)], tgt_vs[b].at[pl.ds(0, 16)],
                         tsems[b])

    def wait_idx(b):
        pltpu.make_async_copy(idxc.at[pl.ds(0, 64)], idxc_vs[b], isems[b]).wait()
        pltpu.make_async_copy(tgt.at[pl.ds(0, 16)], tgt_vs[b].at[pl.ds(0, 16)],
                              tsems[b]).wait()

    def fix_idx(b):
        for q in range(4):
            dq = pl.ds(16 * q, 16)
            idxc_vs[b][dq] = idxc_vs[b][dq] + topic_off

    def dma_gather(b):
        pltpu.async_copy(big.at[idxc_vs[b].at[pl.ds(0, 32)]],
                         big_vs[b].at[pl.ds(0, 32), :], gsems[b])
        pltpu.async_copy(big.at[idxc_vs[b].at[pl.ds(32, 32)]],
                         big_vs[b].at[pl.ds(32, 32), :], hsems[b])
        pltpu.async_copy(ft2.at[idxc_vs[b]], ft2g_vs[b], fsems[b])

    def wait_gather(b):
        pltpu.make_async_copy(big.at[idxc_vs[b].at[pl.ds(0, 32)]],
                              big_vs[b].at[pl.ds(0, 32), :], gsems[b]).wait()
        pltpu.make_async_copy(big.at[idxc_vs[b].at[pl.ds(32, 32)]],
                              big_vs[b].at[pl.ds(32, 32), :], hsems[b]).wait()
        pltpu.make_async_copy(ft2.at[idxc_vs[b]], ft2g_vs[b], fsems[b]).wait()

    def slice_body(r, _):
        k = wid * SPW + r
        t0 = k * TPS

        pltpu.sync_copy(zacc, acc_v)

        def zero_body(t, _):
            den_v[t, :] = zero16
            return 0

        lax.fori_loop(0, TPS, zero_body, 0)

        pltpu.sync_copy(nl.at[pl.ds(t0, TPS)], nl_v)
        pltpu.async_copy(ft2.at[nl_v], a1g_v, gsem0).wait()

        rpv = rp_v[pl.ds(k, 16)]
        e0 = rpv[0]
        e1 = rpv[1]
        c0 = e0 // 16
        c1 = (e1 + 15) // 16

        def compute(c, b):
            base = c * 16
            tgt_b = tgt_vs[b]
            big_b = big_vs[b]
            tvec = tgt_b[pl.ds(0, 16)]
            t_c16 = jnp.minimum(jnp.maximum(tvec - t0, 0), TPS - 1)
            ev = lane + base
            vf = ((ev >= e0) & (ev < e1)).astype(jnp.float32)
            for h in range(H):
                hv = jnp.full((16,), h, jnp.int32)
                a1vec = plsc.load_gather(a1g_v, [t_c16, hv + 4])
                s0 = plsc.load_gather(ft2g_vs[b], [row4, hv])
                s1 = plsc.load_gather(ft2g_vs[b], [row4 + 1, hv])
                s2 = plsc.load_gather(ft2g_vs[b], [row4 + 2, hv])
                s3 = plsc.load_gather(ft2g_vs[b], [row4 + 3, hv])
                a = a1vec + (s0 + s1 + s2) * third + s3
                a = jnp.maximum(a, jnp.float32(0.01) * a)
                g = jnp.exp(a) * vf
                plsc.store_scatter(g_v, [lane, hv], g)

            def flush(t_cur, regs):
                plsc.addupdate(den_v.at[t_cur, :], regs[32])
                for h in range(H):
                    for j in range(8):
                        col = 128 * h + 16 * j
                        plsc.addupdate(acc_v.at[t_cur, pl.ds(col, 16)],
                                       regs[h * 8 + j])

            def edge_body(e, carry):
                t_cur = carry[0]
                regs = carry[1:]
                b4 = e * 4
                t = tgt_b[pl.ds(e, 16)][0] - t0
                t_c = jnp.minimum(jnp.maximum(t, 0), TPS - 1)
                is_new = (t_c != t_cur) & (t_cur >= 0)

                @pl.when(is_new)
                def _():
                    flush(t_cur, regs)

                newv = jnp.zeros((16,), jnp.bool_) | is_new
                grow = g_v[e, :]
                gb = [zero16 + grow[h] for h in range(H)]
                out = [None] * 33
                for j in range(8):
                    dj = pl.ds(16 * j, 16)
                    hj = (big_b[b4, dj] + big_b[b4 + 1, dj] + big_b[b4 + 2, dj]) \
                        * third + big_b[b4 + 3, dj]
                    for h in range(H):
                        c = gb[h] * hj
                        out[h * 8 + j] = jnp.where(newv, c, regs[h * 8 + j] + c)
                dc = grow * lane4f
                out[32] = jnp.where(newv, dc, regs[32] + dc)
                return (t_c,) + tuple(out)

            init = (jnp.int32(-1),) + tuple([zero16] * 33)
            fin = lax.fori_loop(0, 16, edge_body, init)

            @pl.when(fin[0] >= 0)
            def _():
                flush(fin[0], fin[1:])

        dma_idx(c0, 0)
        wait_idx(0)
        fix_idx(0)
        dma_gather(0)
        npairs = (c1 - c0 + 1) // 2

        def pair_body(i, _):
            c = c0 + 2 * i
            dma_idx(c + 1, 1)
            wait_idx(1)
            fix_idx(1)
            wait_gather(0)
            dma_gather(1)
            compute(c, 0)
            dma_idx(c + 2, 0)
            wait_idx(0)
            fix_idx(0)
            wait_gather(1)
            dma_gather(0)
            compute(c + 1, 1)
            return 0

        lax.fori_loop(0, npairs, pair_body, 0)
        wait_gather(0)

        def fin_body(t, _):
            drow = den_v[t, :]
            for h in range(H):
                dspl = zero16 + (drow[h] + jnp.float32(1e-9))
                for j in range(8):
                    col = 128 * h + 16 * j
                    v = acc_v[t, pl.ds(col, 16)] / dspl
                    v = jnp.where(v > 0, v, jnp.exp(v) - jnp.float32(1.0))
                    acc_v[t, pl.ds(col, 16)] = v
            return 0

        lax.fori_loop(0, TPS, fin_body, 0)
        pltpu.sync_copy(acc_v, hp_out.at[pl.ds(t0, TPS), :])
        return 0

    lax.fori_loop(0, SPW, slice_body, 0)


def _sc_agg(big, ft2, idxc, tgt, nl, rp, zacc):
    mesh = plsc.VectorSubcoreMesh(core_axis_name="c", subcore_axis_name="s")
    f = pl.kernel(
        _sc_agg_body,
        out_type=jax.ShapeDtypeStruct((NT, HD), jnp.float32),
        mesh=mesh,
        compiler_params=pltpu.CompilerParams(needs_layout_passes=False,
                                             use_tc_tiling_on_sc=False),
        scratch_types=[
            pltpu.VMEM((TPS, HD), jnp.float32),
            pltpu.VMEM((TPS, 16), jnp.float32),
            pltpu.VMEM((TPS, 8), jnp.float32),
            pltpu.VMEM((TPS,), jnp.int32),
            pltpu.VMEM((128,), jnp.int32),
            pltpu.VMEM((64,), jnp.int32),
            pltpu.VMEM((64,), jnp.int32),
            pltpu.VMEM((32,), jnp.int32),
            pltpu.VMEM((32,), jnp.int32),
            pltpu.VMEM((64, D), jnp.float32),
            pltpu.VMEM((64, D), jnp.float32),
            pltpu.VMEM((64, 8), jnp.float32),
            pltpu.VMEM((64, 8), jnp.float32),
            pltpu.VMEM((16, 16), jnp.float32),
        ] + [pltpu.SemaphoreType.DMA] * 10,
    )
    return f(big, ft2, idxc, tgt, nl, rp, zacc)


def _scores_body(hp_ref, w1_ref, b1_ref, w2_ref, s_ref):
    i = pl.program_id(0)

    @pl.when(i == 0)
    def _():
        s_ref[0, 0] = jnp.float32(0.0)

    z = jnp.tanh(jnp.dot(hp_ref[...], w1_ref[...],
                         preferred_element_type=jnp.float32) + b1_ref[...])
    s_ref[0, 0] += jnp.sum(z * w2_ref[...])


def _scores(hp, fc1_w, fc1_b, fc2_w):
    BS = 512
    nb = NT // BS
    return pl.pallas_call(
        _scores_body,
        grid=(nb,),
        in_specs=[
            pl.BlockSpec((BS, HD), lambda i: (i, 0)),
            pl.BlockSpec((HD, 128), lambda i: (0, 0)),
            pl.BlockSpec((1, 128), lambda i: (0, 0)),
            pl.BlockSpec((1, 128), lambda i: (0, 0)),
        ],
        out_specs=pl.BlockSpec((1, 1), lambda i: (0, 0), memory_space=pltpu.SMEM),
        out_shape=jax.ShapeDtypeStruct((1, 1), jnp.float32),
    )(hp, fc1_w, fc1_b.reshape(1, 128), fc2_w.reshape(1, 128))


def _combine_body(hp0_ref, hp1_ref, wu_ref, bu_ref, s0_ref, s1_ref,
                  hu_ref, lg_ref, beta_ref):
    i = pl.program_id(0)
    dlt = (s1_ref[0, 0] - s0_ref[0, 0]) / jnp.float32(NT)
    b0 = jnp.float32(1.0) / (jnp.float32(1.0) + jnp.exp(dlt))
    b1 = jnp.float32(1.0) - b0

    @pl.when(i == 0)
    def _():
        col = lax.broadcasted_iota(jnp.int32, (1, 128), 1)
        beta_ref[...] = jnp.where(col == 0, b0, jnp.where(col == 1, b1, 0.0))

    hu = b0 * hp0_ref[...] + b1 * hp1_ref[...]
    hu_ref[...] = hu
    lg_ref[...] = jnp.dot(hu, wu_ref[...], preferred_element_type=jnp.float32) \
        + bu_ref[...]


def _combine(hp0, hp1, fc_user_w, fc_user_b, s0, s1):
    BS = 512
    nb = NT // BS
    return pl.pallas_call(
        _combine_body,
        grid=(nb,),
        in_specs=[
            pl.BlockSpec((BS, HD), lambda i: (i, 0)),
            pl.BlockSpec((BS, HD), lambda i: (i, 0)),
            pl.BlockSpec((HD, D), lambda i: (0, 0)),
            pl.BlockSpec((1, D), lambda i: (0, 0)),
            pl.BlockSpec(memory_space=pltpu.SMEM),
            pl.BlockSpec(memory_space=pltpu.SMEM),
        ],
        out_specs=(
            pl.BlockSpec((BS, HD), lambda i: (i, 0)),
            pl.BlockSpec((BS, D), lambda i: (i, 0)),
            pl.BlockSpec((1, 128), lambda i: (0, 0)),
        ),
        out_shape=(
            jax.ShapeDtypeStruct((NT, HD), jnp.float32),
            jax.ShapeDtypeStruct((NT, D), jnp.float32),
            jax.ShapeDtypeStruct((1, 128), jnp.float32),
        ),
    )(hp0, hp1, fc_user_w, fc_user_b.reshape(1, D), s0, s1)


@jax.jit
def kernel(features, topic, type_mask, edge_metapath_indices_0,
           edge_metapath_indices_1, edge_metapath_text_indices_0,
           edge_metapath_text_indices_1, target_idx_0, target_idx_1,
           node_list_0, node_list_1, attn1, attn2, fc1_w, fc1_b, fc2_w,
           fc_user_w, fc_user_b):
    del type_mask
    i32 = jnp.int32
    idxc0 = jnp.concatenate(
        [edge_metapath_indices_0.astype(i32),
         edge_metapath_text_indices_0.astype(i32)[:, None]], axis=1).reshape(-1)
    idxc1 = jnp.concatenate(
        [edge_metapath_indices_1.astype(i32),
         edge_metapath_text_indices_1.astype(i32)[:, None]], axis=1).reshape(-1)
    tgt0 = target_idx_0.astype(i32)
    tgt1 = target_idx_1.astype(i32)
    nl0 = node_list_0.astype(i32)
    nl1 = node_list_1.astype(i32)
    big = jnp.concatenate([features, topic], axis=0)
    w8 = jnp.concatenate([attn2.T, attn1], axis=1)
    zacc = jnp.zeros((TPS, HD), jnp.float32)

    ft2 = _proj(big, w8)
    rp0 = _bounds(tgt0)
    hp0 = _sc_agg(big, ft2, idxc0, tgt0, nl0, rp0, zacc)
    rp1 = _bounds(tgt1)
    hp1 = _sc_agg(big, ft2, idxc1, tgt1, nl1, rp1, zacc)
    s0 = _scores(hp0, fc1_w, fc1_b, fc2_w)
    s1 = _scores(hp1, fc1_w, fc1_b, fc2_w)
    h_user, logits, beta_mat = _combine(hp0, hp1, fc_user_w, fc_user_b, s0, s1)
    return h_user, logits, beta_mat[0, :2]


# speedup vs baseline: 1.7215x; 1.2461x over previous
import functools

import jax
import jax.numpy as jnp
from jax import lax
from jax.experimental import pallas as pl
from jax.experimental.pallas import tpu as pltpu
from jax.experimental.pallas import tpu_sc as plsc

N_NODES = 10000
NT = 8192
E = 160000
L = 3
D = 128
H = 4
HD = H * D
NSLICE = 64
TPS = NT // NSLICE
NWORK = 32
SPW = NSLICE // NWORK
EP_ROWS = 1280


def _bounds_body(t_ref, rp_ref):
    krow = lax.broadcasted_iota(jnp.int32, (128, 128), 0)

    def body(r, acc):
        row = t_ref[pl.ds(r, 1), :]
        sid = row >> 7
        return acc + (krow == sid).astype(jnp.float32)

    hist = lax.fori_loop(0, EP_ROWS, body, jnp.zeros((128, 128), jnp.float32))
    hist_row = jnp.sum(hist, axis=1)[None, :]
    s_ids = lax.broadcasted_iota(jnp.int32, (128, 128), 1)
    mask = (s_ids < krow).astype(jnp.float32)
    rp = jnp.sum(mask * hist_row, axis=1, keepdims=True)
    rp_ref[...] = rp.astype(jnp.int32)


def _bounds(tgt):
    pad = EP_ROWS * 128 - E
    fill = jnp.full((pad,), jnp.int32(1 << 30), jnp.int32)
    t = jnp.concatenate([tgt, fill]).reshape(EP_ROWS, 128)
    rp = pl.pallas_call(
        _bounds_body,
        out_shape=jax.ShapeDtypeStruct((128, 1), jnp.int32),
    )(t)
    return rp.reshape(128)


def _proj_body(big_ref, w8_ref, out_ref):
    out_ref[...] = jnp.dot(big_ref[...], w8_ref[...],
                           preferred_element_type=jnp.float32)


def _proj(big, w8):
    NB = 20
    BS = 2 * N_NODES // NB
    return pl.pallas_call(
        _proj_body,
        grid=(NB,),
        in_specs=[
            pl.BlockSpec((BS, D), lambda i: (i, 0)),
            pl.BlockSpec((D, 8), lambda i: (0, 0)),
        ],
        out_specs=pl.BlockSpec((BS, 8), lambda i: (i, 0)),
        out_shape=jax.ShapeDtypeStruct((2 * N_NODES, 8), jnp.float32),
    )(big, w8)


NCHUNK = E // 16


def _sc_agg_body(big, ft2, idxc, tgt, nl, rp, zacc,
                 hp_out,
                 acc_v, den_v, a1g_v, nl_v, rp_v,
                 idxc_v0, idxc_v1, tgt_v0, tgt_v1, big_v0, big_v1,
                 ft2g_v0, ft2g_v1, g_v, tcur_v,
                 isem0, isem1, tsem0, tsem1, gsem0, gsem1, fsem0, fsem1,
                 hsem0, hsem1):
    cid = lax.axis_index("c")
    sid = lax.axis_index("s")
    wid = cid * 16 + sid

    idxc_vs = (idxc_v0, idxc_v1)
    tgt_vs = (tgt_v0, tgt_v1)
    big_vs = (big_v0, big_v1)
    ft2g_vs = (ft2g_v0, ft2g_v1)
    isems = (isem0, isem1)
    tsems = (tsem0, tsem1)
    gsems = (gsem0, gsem1)
    fsems = (fsem0, fsem1)
    hsems = (hsem0, hsem1)

    pltpu.sync_copy(rp, rp_v)

    lane = lax.broadcasted_iota(jnp.int32, (16,), 0)
    lane4f = (lane < 4).astype(jnp.float32)
    row4 = lane * 4
    topic_off = jnp.where(lane % 4 == 3, N_NODES, 0)
    third = jnp.float32(1.0 / 3.0)
    zero16 = jnp.zeros((16,), jnp.float32)

    def dma_idx(c, b):
        cc = jnp.minimum(c, NCHUNK - 1)
        pltpu.async_copy(idxc.at[pl.ds(cc * 64, 64)], idxc_vs[b], isems[b])
        pltpu.async_copy(tgt.at[pl.ds(cc * 16, 16)], tgt_vs[b].at[pl.ds(0, 16)],
                         tsems[b])

    def wait_idx(b):
        pltpu.make_async_copy(idxc.at[pl.ds(0, 64)], idxc_vs[b], isems[b]).wait()
        pltpu.make_async_copy(tgt.at[pl.ds(0, 16)], tgt_vs[b].at[pl.ds(0, 16)],
                              tsems[b]).wait()

    def fix_idx(b):
        for q in range(4):
            dq = pl.ds(16 * q, 16)
            idxc_vs[b][dq] = idxc_vs[b][dq] + topic_off

    def dma_gather(b):
        pltpu.async_copy(big.at[idxc_vs[b].at[pl.ds(0, 32)]],
                         big_vs[b].at[pl.ds(0, 32), :], gsems[b])
        pltpu.async_copy(big.at[idxc_vs[b].at[pl.ds(32, 32)]],
                         big_vs[b].at[pl.ds(32, 32), :], hsems[b])
        pltpu.async_copy(ft2.at[idxc_vs[b]], ft2g_vs[b], fsems[b])

    def wait_gather(b):
        pltpu.make_async_copy(big.at[idxc_vs[b].at[pl.ds(0, 32)]],
                              big_vs[b].at[pl.ds(0, 32), :], gsems[b]).wait()
        pltpu.make_async_copy(big.at[idxc_vs[b].at[pl.ds(32, 32)]],
                              big_vs[b].at[pl.ds(32, 32), :], hsems[b]).wait()
        pltpu.make_async_copy(ft2.at[idxc_vs[b]], ft2g_vs[b], fsems[b]).wait()

    def slice_body(r, _):
        k = wid * SPW + r
        t0 = k * TPS

        pltpu.sync_copy(zacc, acc_v)

        def zero_body(t, _):
            den_v[t, :] = zero16
            return 0

        lax.fori_loop(0, TPS, zero_body, 0)

        pltpu.sync_copy(nl.at[pl.ds(t0, TPS)], nl_v)
        pltpu.async_copy(ft2.at[nl_v], a1g_v, gsem0).wait()

        rpv = rp_v[pl.ds(k, 16)]
        e0 = rpv[0]
        e1 = rpv[1]
        c0 = e0 // 16
        c1 = (e1 + 15) // 16

        def compute(c, b):
            base = c * 16
            tgt_b = tcur_v
            big_b = big_vs[b]
            tvec = tgt_b[pl.ds(0, 16)]
            t_c16 = jnp.minimum(jnp.maximum(tvec - t0, 0), TPS - 1)
            ev = lane + base
            vf = ((ev >= e0) & (ev < e1)).astype(jnp.float32)
            for h in range(H):
                hv = jnp.full((16,), h, jnp.int32)
                a1vec = plsc.load_gather(a1g_v, [t_c16, hv + 4])
                s0 = plsc.load_gather(ft2g_vs[b], [row4, hv])
                s1 = plsc.load_gather(ft2g_vs[b], [row4 + 1, hv])
                s2 = plsc.load_gather(ft2g_vs[b], [row4 + 2, hv])
                s3 = plsc.load_gather(ft2g_vs[b], [row4 + 3, hv])
                a = a1vec + (s0 + s1 + s2) * third + s3
                a = jnp.maximum(a, jnp.float32(0.01) * a)
                g = jnp.exp(a) * vf
                plsc.store_scatter(g_v, [lane, hv], g)

            def flush(t_cur, regs):
                plsc.addupdate(den_v.at[t_cur, :], regs[32])
                for h in range(H):
                    for j in range(8):
                        col = 128 * h + 16 * j
                        plsc.addupdate(acc_v.at[t_cur, pl.ds(col, 16)],
                                       regs[h * 8 + j])

            def edge_body(e, carry):
                t_cur = carry[0]
                regs = carry[1:]
                b4 = e * 4
                t = tgt_b[pl.ds(e, 16)][0] - t0
                t_c = jnp.minimum(jnp.maximum(t, 0), TPS - 1)
                is_new = (t_c != t_cur) & (t_cur >= 0)

                @pl.when(is_new)
                def _():
                    flush(t_cur, regs)

                newv = jnp.zeros((16,), jnp.bool_) | is_new
                grow = g_v[e, :]
                gb = [zero16 + grow[h] for h in range(H)]
                out = [None] * 33
                for j in range(8):
                    dj = pl.ds(16 * j, 16)
                    hj = (big_b[b4, dj] + big_b[b4 + 1, dj] + big_b[b4 + 2, dj]) \
                        * third + big_b[b4 + 3, dj]
                    for h in range(H):
                        c = gb[h] * hj
                        out[h * 8 + j] = jnp.where(newv, c, regs[h * 8 + j] + c)
                dc = grow * lane4f
                out[32] = jnp.where(newv, dc, regs[32] + dc)
                return (t_c,) + tuple(out)

            init = (jnp.int32(-1),) + tuple([zero16] * 33)
            fin = lax.fori_loop(0, 16, edge_body, init)

            @pl.when(fin[0] >= 0)
            def _():
                flush(fin[0], fin[1:])

        dma_idx(c0, 0)
        wait_idx(0)
        fix_idx(0)
        dma_gather(0)
        dma_idx(c0 + 1, 1)
        npairs = (c1 - c0 + 1) // 2

        def pair_body(i, _):
            c = c0 + 2 * i
            wait_idx(1)
            fix_idx(1)
            wait_gather(0)
            tcur_v[pl.ds(0, 16)] = tgt_vs[0][pl.ds(0, 16)]
            dma_gather(1)
            dma_idx(c + 2, 0)
            compute(c, 0)
            wait_idx(0)
            fix_idx(0)
            wait_gather(1)
            tcur_v[pl.ds(0, 16)] = tgt_vs[1][pl.ds(0, 16)]
            dma_gather(0)
            dma_idx(c + 3, 1)
            compute(c + 1, 1)
            return 0

        lax.fori_loop(0, npairs, pair_body, 0)
        wait_gather(0)
        wait_idx(1)

        def fin_body(t, _):
            drow = den_v[t, :]
            for h in range(H):
                dspl = zero16 + (drow[h] + jnp.float32(1e-9))
                for j in range(8):
                    col = 128 * h + 16 * j
                    v = acc_v[t, pl.ds(col, 16)] / dspl
                    v = jnp.where(v > 0, v, jnp.exp(v) - jnp.float32(1.0))
                    acc_v[t, pl.ds(col, 16)] = v
            return 0

        lax.fori_loop(0, TPS, fin_body, 0)
        pltpu.sync_copy(acc_v, hp_out.at[pl.ds(t0, TPS), :])
        return 0

    lax.fori_loop(0, SPW, slice_body, 0)


def _sc_agg(big, ft2, idxc, tgt, nl, rp, zacc):
    mesh = plsc.VectorSubcoreMesh(core_axis_name="c", subcore_axis_name="s")
    f = pl.kernel(
        _sc_agg_body,
        out_type=jax.ShapeDtypeStruct((NT, HD), jnp.float32),
        mesh=mesh,
        compiler_params=pltpu.CompilerParams(needs_layout_passes=False,
                                             use_tc_tiling_on_sc=False),
        scratch_types=[
            pltpu.VMEM((TPS, HD), jnp.float32),
            pltpu.VMEM((TPS, 16), jnp.float32),
            pltpu.VMEM((TPS, 8), jnp.float32),
            pltpu.VMEM((TPS,), jnp.int32),
            pltpu.VMEM((128,), jnp.int32),
            pltpu.VMEM((64,), jnp.int32),
            pltpu.VMEM((64,), jnp.int32),
            pltpu.VMEM((32,), jnp.int32),
            pltpu.VMEM((32,), jnp.int32),
            pltpu.VMEM((64, D), jnp.float32),
            pltpu.VMEM((64, D), jnp.float32),
            pltpu.VMEM((64, 8), jnp.float32),
            pltpu.VMEM((64, 8), jnp.float32),
            pltpu.VMEM((16, 16), jnp.float32),
            pltpu.VMEM((32,), jnp.int32),
        ] + [pltpu.SemaphoreType.DMA] * 10,
    )
    return f(big, ft2, idxc, tgt, nl, rp, zacc)


def _scores_body(hp_ref, w1_ref, b1_ref, w2_ref, s_ref):
    i = pl.program_id(0)

    @pl.when(i == 0)
    def _():
        s_ref[0, 0] = jnp.float32(0.0)

    z = jnp.tanh(jnp.dot(hp_ref[...], w1_ref[...],
                         preferred_element_type=jnp.float32) + b1_ref[...])
    s_ref[0, 0] += jnp.sum(z * w2_ref[...])


def _scores(hp, fc1_w, fc1_b, fc2_w):
    BS = 512
    nb = NT // BS
    return pl.pallas_call(
        _scores_body,
        grid=(nb,),
        in_specs=[
            pl.BlockSpec((BS, HD), lambda i: (i, 0)),
            pl.BlockSpec((HD, 128), lambda i: (0, 0)),
            pl.BlockSpec((1, 128), lambda i: (0, 0)),
            pl.BlockSpec((1, 128), lambda i: (0, 0)),
        ],
        out_specs=pl.BlockSpec((1, 1), lambda i: (0, 0), memory_space=pltpu.SMEM),
        out_shape=jax.ShapeDtypeStruct((1, 1), jnp.float32),
    )(hp, fc1_w, fc1_b.reshape(1, 128), fc2_w.reshape(1, 128))


def _combine_body(hp0_ref, hp1_ref, wu_ref, bu_ref, s0_ref, s1_ref,
                  hu_ref, lg_ref, beta_ref):
    i = pl.program_id(0)
    dlt = (s1_ref[0, 0] - s0_ref[0, 0]) / jnp.float32(NT)
    b0 = jnp.float32(1.0) / (jnp.float32(1.0) + jnp.exp(dlt))
    b1 = jnp.float32(1.0) - b0

    @pl.when(i == 0)
    def _():
        col = lax.broadcasted_iota(jnp.int32, (1, 128), 1)
        beta_ref[...] = jnp.where(col == 0, b0, jnp.where(col == 1, b1, 0.0))

    hu = b0 * hp0_ref[...] + b1 * hp1_ref[...]
    hu_ref[...] = hu
    lg_ref[...] = jnp.dot(hu, wu_ref[...], preferred_element_type=jnp.float32) \
        + bu_ref[...]


def _combine(hp0, hp1, fc_user_w, fc_user_b, s0, s1):
    BS = 512
    nb = NT // BS
    return pl.pallas_call(
        _combine_body,
        grid=(nb,),
        in_specs=[
            pl.BlockSpec((BS, HD), lambda i: (i, 0)),
            pl.BlockSpec((BS, HD), lambda i: (i, 0)),
            pl.BlockSpec((HD, D), lambda i: (0, 0)),
            pl.BlockSpec((1, D), lambda i: (0, 0)),
            pl.BlockSpec(memory_space=pltpu.SMEM),
            pl.BlockSpec(memory_space=pltpu.SMEM),
        ],
        out_specs=(
            pl.BlockSpec((BS, HD), lambda i: (i, 0)),
            pl.BlockSpec((BS, D), lambda i: (i, 0)),
            pl.BlockSpec((1, 128), lambda i: (0, 0)),
        ),
        out_shape=(
            jax.ShapeDtypeStruct((NT, HD), jnp.float32),
            jax.ShapeDtypeStruct((NT, D), jnp.float32),
            jax.ShapeDtypeStruct((1, 128), jnp.float32),
        ),
    )(hp0, hp1, fc_user_w, fc_user_b.reshape(1, D), s0, s1)


@jax.jit
def kernel(features, topic, type_mask, edge_metapath_indices_0,
           edge_metapath_indices_1, edge_metapath_text_indices_0,
           edge_metapath_text_indices_1, target_idx_0, target_idx_1,
           node_list_0, node_list_1, attn1, attn2, fc1_w, fc1_b, fc2_w,
           fc_user_w, fc_user_b):
    del type_mask
    i32 = jnp.int32
    idxc0 = jnp.concatenate(
        [edge_metapath_indices_0.astype(i32),
         edge_metapath_text_indices_0.astype(i32)[:, None]], axis=1).reshape(-1)
    idxc1 = jnp.concatenate(
        [edge_metapath_indices_1.astype(i32),
         edge_metapath_text_indices_1.astype(i32)[:, None]], axis=1).reshape(-1)
    tgt0 = target_idx_0.astype(i32)
    tgt1 = target_idx_1.astype(i32)
    nl0 = node_list_0.astype(i32)
    nl1 = node_list_1.astype(i32)
    big = jnp.concatenate([features, topic], axis=0)
    w8 = jnp.concatenate([attn2.T, attn1], axis=1)
    zacc = jnp.zeros((TPS, HD), jnp.float32)

    ft2 = _proj(big, w8)
    rp0 = _bounds(tgt0)
    hp0 = _sc_agg(big, ft2, idxc0, tgt0, nl0, rp0, zacc)
    rp1 = _bounds(tgt1)
    hp1 = _sc_agg(big, ft2, idxc1, tgt1, nl1, rp1, zacc)
    s0 = _scores(hp0, fc1_w, fc1_b, fc2_w)
    s1 = _scores(hp1, fc1_w, fc1_b, fc2_w)
    h_user, logits, beta_mat = _combine(hp0, hp1, fc_user_w, fc_user_b, s0, s1)
    return h_user, logits, beta_mat[0, :2]
